# Initial kernel scaffold; baseline (speedup 1.0000x reference)
import functools

import jax
import jax.numpy as jnp
from jax.experimental import pallas as pl
from jax.experimental.pallas import tpu as pltpu

EMB = 2048
NUM_EXPERTS = 64
TOP_K = 8
SEQ_AUX_ALPHA = 0.001

ROWS = 1024


def _gate_kernel(x_ref, w_ref, b_ref, idx_ref, wgt_ref, loss_ref,
                 f_acc, p_acc, *, tiles_per_batch, num_tiles, seq_len):
    step = pl.program_id(0)

    @pl.when(step == 0)
    def _init():
        f_acc[...] = jnp.zeros_like(f_acc)
        p_acc[...] = jnp.zeros_like(p_acc)

    logits = jax.lax.dot_general(
        x_ref[...], w_ref[...],
        dimension_numbers=(((1,), (1,)), ((), ())),
        preferred_element_type=jnp.float32)
    scores = jax.nn.sigmoid(logits)
    biased = logits + b_ref[...]

    rows = logits.shape[0]
    iota = jax.lax.broadcasted_iota(jnp.int32, (rows, NUM_EXPERTS), 1)

    masked = biased
    cnt = jnp.zeros((rows, NUM_EXPERTS), jnp.float32)
    idx_cols = []
    wgt_cols = []
    for _ in range(TOP_K):
        m = jnp.max(masked, axis=1, keepdims=True)
        idx = jnp.min(jnp.where(masked == m, iota, NUM_EXPERTS),
                      axis=1, keepdims=True)
        sel = iota == idx
        idx_cols.append(idx)
        wgt_cols.append(jnp.sum(jnp.where(sel, scores, 0.0),
                                axis=1, keepdims=True))
        masked = jnp.where(sel, -jnp.inf, masked)
        cnt += sel.astype(jnp.float32)

    idx_ref[...] = jnp.concatenate(idx_cols, axis=1)
    wgt = jnp.concatenate(wgt_cols, axis=1)
    wgt_ref[...] = wgt / (jnp.sum(wgt, axis=1, keepdims=True) + 1e-10)

    batch = step // tiles_per_batch
    rs = jnp.sum(scores, axis=1, keepdims=True) + 1e-10
    p_part = jnp.sum(scores / rs, axis=0, keepdims=True)
    f_part = jnp.sum(cnt, axis=0, keepdims=True)
    biota = jax.lax.broadcasted_iota(jnp.int32, (f_acc.shape[0], 1), 0)
    onb = (biota == batch).astype(jnp.float32)
    f_acc[...] += onb * f_part
    p_acc[...] += onb * p_part

    @pl.when(step == num_tiles - 1)
    def _fin():
        num_batches = f_acc.shape[0]
        scale = SEQ_AUX_ALPHA / (num_batches * TOP_K * seq_len * seq_len)
        loss_ref[0, 0] = jnp.sum(f_acc[...] * p_acc[...]) * scale


def kernel(x, weight, expert_bias):
    bsz, seq_len, emb = x.shape
    x_flat = x.reshape(-1, emb)
    n = x_flat.shape[0]
    num_tiles = n // ROWS
    tiles_per_batch = seq_len // ROWS

    body = functools.partial(
        _gate_kernel, tiles_per_batch=tiles_per_batch,
        num_tiles=num_tiles, seq_len=seq_len)

    idx, wgt, loss = pl.pallas_call(
        body,
        grid=(num_tiles,),
        in_specs=[
            pl.BlockSpec((ROWS, emb), lambda i: (i, 0)),
            pl.BlockSpec((NUM_EXPERTS, emb), lambda i: (0, 0)),
            pl.BlockSpec((1, NUM_EXPERTS), lambda i: (0, 0)),
        ],
        out_specs=[
            pl.BlockSpec((ROWS, TOP_K), lambda i: (i, 0)),
            pl.BlockSpec((ROWS, TOP_K), lambda i: (i, 0)),
            pl.BlockSpec((1, 1), lambda i: (0, 0)),
        ],
        out_shape=[
            jax.ShapeDtypeStruct((n, TOP_K), jnp.int32),
            jax.ShapeDtypeStruct((n, TOP_K), jnp.float32),
            jax.ShapeDtypeStruct((1, 1), jnp.float32),
        ],
        scratch_shapes=[
            pltpu.VMEM((bsz, NUM_EXPERTS), jnp.float32),
            pltpu.VMEM((bsz, NUM_EXPERTS), jnp.float32),
        ],
    )(x_flat, weight, expert_bias.reshape(1, NUM_EXPERTS))

    return idx, wgt, loss[0, 0]


# speedup vs baseline: 1.7054x; 1.7054x over previous
import functools

import jax
import jax.numpy as jnp
from jax.experimental import pallas as pl
from jax.experimental.pallas import tpu as pltpu

EMB = 2048
NUM_EXPERTS = 64
TOP_K = 8
SEQ_AUX_ALPHA = 0.001

ROWS = 1024


def _gate_kernel(x_ref, w_ref, b_ref, idx_ref, wgt_ref, loss_ref,
                 f_acc, p_acc, *, tiles_per_batch, num_tiles, seq_len):
    step = pl.program_id(0)

    @pl.when(step == 0)
    def _init():
        f_acc[...] = jnp.zeros_like(f_acc)
        p_acc[...] = jnp.zeros_like(p_acc)

    logits = jax.lax.dot_general(
        x_ref[...], w_ref[...],
        dimension_numbers=(((1,), (1,)), ((), ())),
        preferred_element_type=jnp.float32)
    scores = jax.nn.sigmoid(logits)
    biased = logits + b_ref[...]

    rows = logits.shape[0]
    iota = jax.lax.broadcasted_iota(jnp.int32, (rows, NUM_EXPERTS), 1)

    masked = biased
    cnt = jnp.zeros((rows, NUM_EXPERTS), jnp.float32)
    idx_cols = []
    wgt_cols = []
    for _ in range(TOP_K):
        m = jnp.max(masked, axis=1, keepdims=True)
        idx = jnp.min(jnp.where(masked == m, iota, NUM_EXPERTS),
                      axis=1, keepdims=True)
        sel = iota == idx
        idx_cols.append(idx)
        wgt_cols.append(jnp.sum(jnp.where(sel, scores, 0.0),
                                axis=1, keepdims=True))
        masked = jnp.where(sel, -jnp.inf, masked)
        cnt += sel.astype(jnp.float32)

    idx_ref[...] = jnp.concatenate(idx_cols, axis=1)
    wgt = jnp.concatenate(wgt_cols, axis=1)
    wgt_ref[...] = wgt / (jnp.sum(wgt, axis=1, keepdims=True) + 1e-10)

    batch = step // tiles_per_batch
    rs = jnp.sum(scores, axis=1, keepdims=True) + 1e-10
    p_part = jnp.sum(scores / rs, axis=0, keepdims=True)
    f_part = jnp.sum(cnt, axis=0, keepdims=True)
    biota = jax.lax.broadcasted_iota(jnp.int32, (f_acc.shape[0], 1), 0)
    onb = (biota == batch).astype(jnp.float32)
    f_acc[...] += onb * f_part
    p_acc[...] += onb * p_part

    @pl.when(step == num_tiles - 1)
    def _fin():
        num_batches = f_acc.shape[0]
        scale = SEQ_AUX_ALPHA / (num_batches * TOP_K * seq_len * seq_len)
        loss_ref[...] = (jnp.sum(f_acc[...] * p_acc[...]) * scale).reshape(1, 1)


def kernel(x, weight, expert_bias):
    bsz, seq_len, emb = x.shape
    x_flat = x.reshape(-1, emb)
    n = x_flat.shape[0]
    num_tiles = n // ROWS
    tiles_per_batch = seq_len // ROWS

    body = functools.partial(
        _gate_kernel, tiles_per_batch=tiles_per_batch,
        num_tiles=num_tiles, seq_len=seq_len)

    idx, wgt, loss = pl.pallas_call(
        body,
        grid=(num_tiles,),
        in_specs=[
            pl.BlockSpec((ROWS, emb), lambda i: (i, 0)),
            pl.BlockSpec((NUM_EXPERTS, emb), lambda i: (0, 0)),
            pl.BlockSpec((1, NUM_EXPERTS), lambda i: (0, 0)),
        ],
        out_specs=[
            pl.BlockSpec((ROWS, TOP_K), lambda i: (i, 0)),
            pl.BlockSpec((ROWS, TOP_K), lambda i: (i, 0)),
            pl.BlockSpec((1, 1), lambda i: (0, 0)),
        ],
        out_shape=[
            jax.ShapeDtypeStruct((n, TOP_K), jnp.int32),
            jax.ShapeDtypeStruct((n, TOP_K), jnp.float32),
            jax.ShapeDtypeStruct((1, 1), jnp.float32),
        ],
        scratch_shapes=[
            pltpu.VMEM((bsz, NUM_EXPERTS), jnp.float32),
            pltpu.VMEM((bsz, NUM_EXPERTS), jnp.float32),
        ],
    )(x_flat, weight, expert_bias.reshape(1, NUM_EXPERTS))

    return idx, wgt, loss[0, 0]


# speedup vs baseline: 3.6488x; 2.1396x over previous
import functools

import jax
import jax.numpy as jnp
from jax.experimental import pallas as pl
from jax.experimental.pallas import tpu as pltpu

EMB = 2048
NUM_EXPERTS = 64
TOP_K = 8
SEQ_AUX_ALPHA = 0.001

ROWS = 1024


def _gate_kernel(x_ref, w_ref, b_ref, idx_ref, wgt_ref, loss_ref,
                 f_acc, p_acc, *, tiles_per_batch, num_tiles, seq_len):
    step = pl.program_id(0)

    @pl.when(step == 0)
    def _init():
        f_acc[...] = jnp.zeros_like(f_acc)
        p_acc[...] = jnp.zeros_like(p_acc)

    logits = jax.lax.dot_general(
        w_ref[...], x_ref[...],
        dimension_numbers=(((1,), (1,)), ((), ())),
        preferred_element_type=jnp.float32)
    scores = jax.nn.sigmoid(logits)
    biased = logits + b_ref[...]

    rows = logits.shape[1]
    iota = jax.lax.broadcasted_iota(jnp.int32, (NUM_EXPERTS, rows), 0)

    masked = biased
    cnt = jnp.zeros((NUM_EXPERTS, rows), jnp.float32)
    idx_rows = []
    wgt_rows = []
    for _ in range(TOP_K):
        m = jnp.max(masked, axis=0, keepdims=True)
        idx = jnp.min(jnp.where(masked == m, iota, NUM_EXPERTS),
                      axis=0, keepdims=True)
        sel = iota == idx
        idx_rows.append(idx)
        wgt_rows.append(jnp.sum(jnp.where(sel, scores, 0.0),
                                axis=0, keepdims=True))
        masked = jnp.where(sel, -jnp.inf, masked)
        cnt += sel.astype(jnp.float32)

    idx_ref[...] = jnp.concatenate(idx_rows, axis=0)
    wgt = jnp.concatenate(wgt_rows, axis=0)
    wgt_ref[...] = wgt / (jnp.sum(wgt, axis=0, keepdims=True) + 1e-10)

    batch = step // tiles_per_batch
    rs = jnp.sum(scores, axis=0, keepdims=True) + 1e-10
    p_part = jnp.sum(scores / rs, axis=1, keepdims=True)
    f_part = jnp.sum(cnt, axis=1, keepdims=True)
    biota = jax.lax.broadcasted_iota(jnp.int32, (1, f_acc.shape[1]), 1)
    onb = (biota == batch).astype(jnp.float32)
    f_acc[...] += f_part * onb
    p_acc[...] += p_part * onb

    @pl.when(step == num_tiles - 1)
    def _fin():
        num_batches = f_acc.shape[1]
        scale = SEQ_AUX_ALPHA / (num_batches * TOP_K * seq_len * seq_len)
        loss_ref[...] = (jnp.sum(f_acc[...] * p_acc[...]) * scale).reshape(1, 1)


def kernel(x, weight, expert_bias):
    bsz, seq_len, emb = x.shape
    x_flat = x.reshape(-1, emb)
    n = x_flat.shape[0]
    num_tiles = n // ROWS
    tiles_per_batch = seq_len // ROWS

    body = functools.partial(
        _gate_kernel, tiles_per_batch=tiles_per_batch,
        num_tiles=num_tiles, seq_len=seq_len)

    idx_t, wgt_t, loss = pl.pallas_call(
        body,
        grid=(num_tiles,),
        in_specs=[
            pl.BlockSpec((ROWS, emb), lambda i: (i, 0)),
            pl.BlockSpec((NUM_EXPERTS, emb), lambda i: (0, 0)),
            pl.BlockSpec((NUM_EXPERTS, 1), lambda i: (0, 0)),
        ],
        out_specs=[
            pl.BlockSpec((TOP_K, ROWS), lambda i: (0, i)),
            pl.BlockSpec((TOP_K, ROWS), lambda i: (0, i)),
            pl.BlockSpec((1, 1), lambda i: (0, 0)),
        ],
        out_shape=[
            jax.ShapeDtypeStruct((TOP_K, n), jnp.int32),
            jax.ShapeDtypeStruct((TOP_K, n), jnp.float32),
            jax.ShapeDtypeStruct((1, 1), jnp.float32),
        ],
        scratch_shapes=[
            pltpu.VMEM((NUM_EXPERTS, bsz), jnp.float32),
            pltpu.VMEM((NUM_EXPERTS, bsz), jnp.float32),
        ],
    )(x_flat, weight, expert_bias.reshape(NUM_EXPERTS, 1))

    return idx_t.T, wgt_t.T, loss[0, 0]


# speedup vs baseline: 3.8433x; 1.0533x over previous
import functools

import jax
import jax.numpy as jnp
from jax.experimental import pallas as pl
from jax.experimental.pallas import tpu as pltpu

EMB = 2048
NUM_EXPERTS = 64
TOP_K = 8
SEQ_AUX_ALPHA = 0.001

ROWS = 2048


def _gate_kernel(x_ref, w_ref, b_ref, idx_ref, wgt_ref, loss_ref,
                 f_acc, p_acc, *, tiles_per_batch, num_tiles, seq_len):
    step = pl.program_id(0)

    @pl.when(step == 0)
    def _init():
        f_acc[...] = jnp.zeros_like(f_acc)
        p_acc[...] = jnp.zeros_like(p_acc)

    logits = jax.lax.dot_general(
        w_ref[...], x_ref[...],
        dimension_numbers=(((1,), (1,)), ((), ())),
        preferred_element_type=jnp.float32)
    scores = jax.nn.sigmoid(logits)
    biased = logits + b_ref[...]

    rows = logits.shape[1]
    iota = jax.lax.broadcasted_iota(jnp.int32, (NUM_EXPERTS, rows), 0)

    masked = biased
    cnt = jnp.zeros((NUM_EXPERTS, rows), jnp.float32)
    idx_rows = []
    wgt_rows = []
    for _ in range(TOP_K):
        m = jnp.max(masked, axis=0, keepdims=True)
        idx = jnp.min(jnp.where(masked == m, iota, NUM_EXPERTS),
                      axis=0, keepdims=True)
        sel = iota == idx
        idx_rows.append(idx)
        wgt_rows.append(jnp.sum(jnp.where(sel, scores, 0.0),
                                axis=0, keepdims=True))
        masked = jnp.where(sel, -jnp.inf, masked)
        cnt += sel.astype(jnp.float32)

    idx_ref[...] = jnp.concatenate(idx_rows, axis=0)
    wgt = jnp.concatenate(wgt_rows, axis=0)
    wgt_ref[...] = wgt / (jnp.sum(wgt, axis=0, keepdims=True) + 1e-10)

    batch = step // tiles_per_batch
    rs = jnp.sum(scores, axis=0, keepdims=True) + 1e-10
    p_part = jnp.sum(scores / rs, axis=1, keepdims=True)
    f_part = jnp.sum(cnt, axis=1, keepdims=True)
    biota = jax.lax.broadcasted_iota(jnp.int32, (1, f_acc.shape[1]), 1)
    onb = (biota == batch).astype(jnp.float32)
    f_acc[...] += f_part * onb
    p_acc[...] += p_part * onb

    @pl.when(step == num_tiles - 1)
    def _fin():
        num_batches = f_acc.shape[1]
        scale = SEQ_AUX_ALPHA / (num_batches * TOP_K * seq_len * seq_len)
        loss_ref[...] = (jnp.sum(f_acc[...] * p_acc[...]) * scale).reshape(1, 1)


def kernel(x, weight, expert_bias):
    bsz, seq_len, emb = x.shape
    x_flat = x.reshape(-1, emb)
    n = x_flat.shape[0]
    num_tiles = n // ROWS
    tiles_per_batch = seq_len // ROWS

    body = functools.partial(
        _gate_kernel, tiles_per_batch=tiles_per_batch,
        num_tiles=num_tiles, seq_len=seq_len)

    idx_t, wgt_t, loss = pl.pallas_call(
        body,
        grid=(num_tiles,),
        in_specs=[
            pl.BlockSpec((ROWS, emb), lambda i: (i, 0)),
            pl.BlockSpec((NUM_EXPERTS, emb), lambda i: (0, 0)),
            pl.BlockSpec((NUM_EXPERTS, 1), lambda i: (0, 0)),
        ],
        out_specs=[
            pl.BlockSpec((TOP_K, ROWS), lambda i: (0, i)),
            pl.BlockSpec((TOP_K, ROWS), lambda i: (0, i)),
            pl.BlockSpec((1, 1), lambda i: (0, 0)),
        ],
        out_shape=[
            jax.ShapeDtypeStruct((TOP_K, n), jnp.int32),
            jax.ShapeDtypeStruct((TOP_K, n), jnp.float32),
            jax.ShapeDtypeStruct((1, 1), jnp.float32),
        ],
        scratch_shapes=[
            pltpu.VMEM((NUM_EXPERTS, bsz), jnp.float32),
            pltpu.VMEM((NUM_EXPERTS, bsz), jnp.float32),
        ],
    )(x_flat, weight, expert_bias.reshape(NUM_EXPERTS, 1))

    return idx_t.T, wgt_t.T, loss[0, 0]
